# Initial kernel scaffold; baseline (speedup 1.0000x reference)
import functools

import jax
import jax.numpy as jnp
from jax import lax
from jax.experimental import pallas as pl
from jax.experimental.pallas import tpu as pltpu
from jax.experimental.pallas import tpu_sc as plsc
from jax.scipy.linalg import block_diag

N = 10000
E = 320000
ET = 2 * E + N
NT = 32
NPT = 320
NP = NT * NPT
B = 128
EPAD = ET + B
NEG_SLOPE = 0.2
NEG_INF = float("-inf")


def _gat16(x, idx):
    dn = lax.GatherDimensionNumbers(
        offset_dims=(), collapsed_slice_dims=(0,), start_index_map=(0,))
    return lax.gather(x, idx[:, None], dn, slice_sizes=(1,),
                      mode=lax.GatherScatterMode.PROMISE_IN_BOUNDS)


def _splat16(x, h):
    return _gat16(x, jnp.full((16,), h, jnp.int32))


def _dense_body(h_ref, w_ref, asrc_ref, adst_ref, feat_ref, el_ref, er_ref,
                *, normalize):
    hb = h_ref[...]
    if normalize:
        s = jnp.sum(hb, axis=1, keepdims=True)
        hb = hb / jnp.maximum(s, 1.0)
    feat = jnp.dot(hb, w_ref[...], preferred_element_type=jnp.float32)
    feat_ref[...] = feat
    el_ref[...] = jnp.dot(feat, asrc_ref[...], preferred_element_type=jnp.float32)
    er_ref[...] = jnp.dot(feat, adst_ref[...], preferred_element_type=jnp.float32)


def _dense(h, W, Asrc, Adst, normalize):
    n, fi = h.shape
    fo = W.shape[1]
    R = 1000
    return pl.pallas_call(
        functools.partial(_dense_body, normalize=normalize),
        grid=(n // R,),
        in_specs=[
            pl.BlockSpec((R, fi), lambda i: (i, 0)),
            pl.BlockSpec((fi, fo), lambda i: (0, 0)),
            pl.BlockSpec((fo, 16), lambda i: (0, 0)),
            pl.BlockSpec((fo, 16), lambda i: (0, 0)),
        ],
        out_specs=[
            pl.BlockSpec((R, fo), lambda i: (i, 0)),
            pl.BlockSpec((R, 16), lambda i: (i, 0)),
            pl.BlockSpec((R, 16), lambda i: (i, 0)),
        ],
        out_shape=[
            jax.ShapeDtypeStruct((n, fo), jnp.float32),
            jax.ShapeDtypeStruct((n, 16), jnp.float32),
            jax.ShapeDtypeStruct((n, 16), jnp.float32),
        ],
    )(h, W, Asrc, Adst)


def _lsm_body(z_ref, out_ref):
    z = z_ref[...]
    col = lax.broadcasted_iota(jnp.int32, z.shape, 1)
    valid = col < 40
    zm = jnp.where(valid, z, NEG_INF)
    m = jnp.max(zm, axis=1, keepdims=True)
    ex = jnp.where(valid, jnp.exp(z - m), 0.0)
    s = jnp.sum(ex, axis=1, keepdims=True)
    res = z - m - jnp.log(s)
    out_ref[...] = res[:, :40]


def _log_softmax(z):
    n = z.shape[0]
    R = 1000
    return pl.pallas_call(
        _lsm_body,
        grid=(n // R,),
        in_specs=[pl.BlockSpec((R, 48), lambda i: (i, 0))],
        out_specs=pl.BlockSpec((R, 40), lambda i: (i, 0)),
        out_shape=jax.ShapeDtypeStruct((n, 40), jnp.float32),
    )(z)


def _make_edge(H, F, activate):
    NCH = F // 16
    hmap = [min((16 * i) // 64, H - 1) for i in range(NCH)]
    mesh = plsc.VectorSubcoreMesh(core_axis_name="c", subcore_axis_name="s",
                                  num_cores=2, num_subcores=16)

    @functools.partial(
        pl.kernel,
        out_type=jax.ShapeDtypeStruct((NP, F), jnp.float32),
        mesh=mesh,
        scratch_types=[
            pltpu.VMEM((NPT, F), jnp.float32),
            pltpu.VMEM((NPT, 16), jnp.float32),
            pltpu.VMEM((B,), jnp.int32),
            pltpu.VMEM((B,), jnp.int32),
            pltpu.SMEM((B,), jnp.int32),
            pltpu.VMEM((B, 16), jnp.float32),
            pltpu.VMEM((B, 16), jnp.float32),
            pltpu.VMEM((B, F), jnp.float32),
            pltpu.SMEM((40,), jnp.int32),
            pltpu.SemaphoreType.DMA,
        ],
    )
    def edge_kernel(src_hbm, dst_hbm, el_hbm, er_hbm, feat_hbm, bnd_hbm,
                    out_hbm, acc, md, srcb, dstb, dstbs, elb, erb, featb,
                    bsm, sem):
        lane = lax.broadcasted_iota(jnp.int32, (16,), 0)
        wid = lax.axis_index("c") * 16 + lax.axis_index("s")
        base = wid * NPT
        pltpu.sync_copy(bnd_hbm, bsm)
        start = bsm[wid]
        end = bsm[wid + 1]
        start8 = (start // 8) * 8
        nblk = (end - start8 + B - 1) // B
        zero16 = jnp.zeros((16,), jnp.float32)
        mdinit = jnp.where(lane < 8, NEG_INF, 0.0)
        emask = lane < 8
        dmask = jnp.logical_and(lane >= 8, lane < 8 + H)
        ci = jnp.array([0] * 8 + [0, 1, 2, 3] + [0] * 4, jnp.int32)

        def init_node(n0, _):
            for i in range(NCH):
                acc[n0, pl.ds(16 * i, 16)] = zero16
            md[n0, :] = mdinit
            return 0
        lax.fori_loop(0, NPT, init_node, 0)

        def load_block(e0, with_feat):
            pltpu.sync_copy(src_hbm.at[pl.ds(e0, B)], srcb)
            pltpu.sync_copy(dst_hbm.at[pl.ds(e0, B)], dstb)
            pltpu.sync_copy(dst_hbm.at[pl.ds(e0, B)], dstbs)
            pltpu.async_copy(el_hbm.at[srcb], elb, sem).wait()
            pltpu.async_copy(er_hbm.at[dstb], erb, sem).wait()
            if with_feat:
                pltpu.async_copy(feat_hbm.at[srcb], featb, sem).wait()

        def b1_blk(k, _):
            e0 = start8 + k * B

            def b1_edge(j, _):
                eid = e0 + j
                valid = jnp.logical_and(eid >= start, eid < end)
                dcl = jnp.clip(dstbs[j] - base, 0, NPT - 1)
                s16 = elb[j, :] + erb[j, :]
                e16 = jnp.where(s16 >= 0, s16, NEG_SLOPE * s16)
                row = md[dcl, :]
                upd = jnp.logical_and(emask, valid)
                md[dcl, :] = jnp.where(upd, jnp.maximum(row, e16), row)
                return 0

            load_block(e0, False)
            lax.fori_loop(0, B, b1_edge, 0)
            return 0
        lax.fori_loop(0, nblk, b1_blk, 0)

        def c_blk(k, carry):
            e0 = start8 + k * B

            def c_edge(j, carry):
                d_prev, regs = carry
                eid = e0 + j
                valid = jnp.logical_and(eid >= start, eid < end)
                dcl = jnp.clip(dstbs[j] - base, 0, NPT - 1)
                d = jnp.where(valid, dcl, d_prev)
                flush = d != d_prev

                @pl.when(flush)
                def _():
                    for i in range(NCH):
                        sl = pl.ds(16 * i, 16)
                        acc[d_prev, sl] = acc[d_prev, sl] + regs[i]

                regs = tuple(jnp.where(flush, zero16, r) for r in regs)
                s16 = elb[j, :] + erb[j, :]
                e16 = jnp.where(s16 >= 0, s16, NEG_SLOPE * s16)
                row = md[d, :]
                ex16 = jnp.where(valid, jnp.exp(e16 - row), zero16)
                md[d, :] = row + jnp.where(dmask, _gat16(ex16, ci), zero16)
                exsp = [_splat16(ex16, h) for h in range(H)]
                new_regs = tuple(
                    regs[i] + exsp[hmap[i]] * featb[j, pl.ds(16 * i, 16)]
                    for i in range(NCH))
                return d, new_regs

            load_block(e0, True)
            return lax.fori_loop(0, B, c_edge, carry)

        carry0 = (jnp.int32(0), tuple(zero16 for _ in range(NCH)))
        d_prev, regs = lax.fori_loop(0, nblk, c_blk, carry0)
        for i in range(NCH):
            sl = pl.ds(16 * i, 16)
            acc[d_prev, sl] = acc[d_prev, sl] + regs[i]

        def d_node(n0, _):
            row = md[n0, :]
            rs = [1.0 / jnp.maximum(_splat16(row, 8 + h), 1e-9)
                  for h in range(H)]
            for i in range(NCH):
                sl = pl.ds(16 * i, 16)
                v = acc[n0, sl] * rs[hmap[i]]
                if activate:
                    v = jnp.where(v > 0, v, jnp.exp(v) - 1.0)
                acc[n0, sl] = v
            return 0
        lax.fori_loop(0, NPT, d_node, 0)
        pltpu.sync_copy(acc, out_hbm.at[pl.ds(base, NPT), :])

    return edge_kernel


_edge256 = _make_edge(4, 256, True)
_edge48 = _make_edge(1, 48, False)


def _amat(a, fo):
    ab = block_diag(*[a[h][:, None] for h in range(a.shape[0])])
    return jnp.pad(ab, ((0, fo - ab.shape[0]), (0, 16 - ab.shape[1])))


def kernel(x, edge_index, pred, conf, lg_s, node_s, epoch,
           W0, a0_src, a0_dst, W1, a1_src, a1_dst, W2, a2_src, a2_dst):
    src0 = edge_index[0].astype(jnp.int32)
    dst0 = edge_index[1].astype(jnp.int32)
    loops = jnp.arange(N, dtype=jnp.int32)
    src = jnp.concatenate([src0, dst0, loops])
    dst = jnp.concatenate([dst0, src0, loops])
    dst_s, src_s = lax.sort((dst, src), num_keys=1)
    bounds = jnp.searchsorted(
        dst_s, jnp.arange(33, dtype=jnp.int32) * NPT).astype(jnp.int32)
    bounds = jnp.concatenate([bounds, jnp.full((7,), ET, jnp.int32)])
    pad = jnp.zeros((EPAD - ET,), jnp.int32)
    srcp = jnp.concatenate([src_s, pad])
    dstp = jnp.concatenate([dst_s, pad])

    A0s, A0d = _amat(a0_src, 256), _amat(a0_dst, 256)
    A1s, A1d = _amat(a1_src, 256), _amat(a1_dst, 256)
    A2s, A2d = _amat(a2_src, 48), _amat(a2_dst, 48)
    W2p = jnp.pad(W2, ((0, 0), (0, 8)))

    feat0, el0, er0 = _dense(x, W0, A0s, A0d, True)
    o0 = _edge256(srcp, dstp, el0, er0, feat0, bounds)
    feat1, el1, er1 = _dense(o0[:N], W1, A1s, A1d, False)
    o1 = _edge256(srcp, dstp, el1, er1, feat1, bounds)
    feat2, el2, er2 = _dense(o1[:N], W2p, A2s, A2d, False)
    o2 = _edge48(srcp, dstp, el2, er2, feat2, bounds)
    return _log_softmax(o2[:N])


# speedup vs baseline: 23.5082x; 23.5082x over previous
import functools

import jax
import jax.numpy as jnp
from jax import lax
from jax.experimental import pallas as pl
from jax.experimental.pallas import tpu as pltpu
from jax.experimental.pallas import tpu_sc as plsc
from jax.scipy.linalg import block_diag

N = 10000
E = 320000
ET = 2 * E + N
NT = 32
NPT = 320
NP = NT * NPT
B = 128
EPAD = ET + B
NEG_SLOPE = 0.2
NEG_INF = float("-inf")


def _gat16(x, idx):
    dn = lax.GatherDimensionNumbers(
        offset_dims=(), collapsed_slice_dims=(0,), start_index_map=(0,))
    return lax.gather(x, idx[:, None], dn, slice_sizes=(1,),
                      mode=lax.GatherScatterMode.PROMISE_IN_BOUNDS)


def _splat16(x, h):
    idx = lax.broadcasted_iota(jnp.int32, (16,), 0) * 0 + h
    return _gat16(x, idx)


def _dense_body(h_ref, w_ref, asrc_ref, adst_ref, feat_ref, el_ref, er_ref,
                *, normalize):
    hb = h_ref[...]
    if normalize:
        s = jnp.sum(hb, axis=1, keepdims=True)
        hb = hb / jnp.maximum(s, 1.0)
    feat = jnp.dot(hb, w_ref[...], preferred_element_type=jnp.float32)
    feat_ref[...] = feat
    el_ref[...] = jnp.dot(feat, asrc_ref[...], preferred_element_type=jnp.float32)
    er_ref[...] = jnp.dot(feat, adst_ref[...], preferred_element_type=jnp.float32)


def _dense(h, W, Asrc, Adst, normalize):
    n, fi = h.shape
    fo = W.shape[1]
    R = 1000
    return pl.pallas_call(
        functools.partial(_dense_body, normalize=normalize),
        grid=(n // R,),
        in_specs=[
            pl.BlockSpec((R, fi), lambda i: (i, 0)),
            pl.BlockSpec((fi, fo), lambda i: (0, 0)),
            pl.BlockSpec((fo, 16), lambda i: (0, 0)),
            pl.BlockSpec((fo, 16), lambda i: (0, 0)),
        ],
        out_specs=[
            pl.BlockSpec((R, fo), lambda i: (i, 0)),
            pl.BlockSpec((R, 16), lambda i: (i, 0)),
            pl.BlockSpec((R, 16), lambda i: (i, 0)),
        ],
        out_shape=[
            jax.ShapeDtypeStruct((n, fo), jnp.float32),
            jax.ShapeDtypeStruct((n, 16), jnp.float32),
            jax.ShapeDtypeStruct((n, 16), jnp.float32),
        ],
    )(h, W, Asrc, Adst)


def _lsm_body(z_ref, out_ref):
    z = z_ref[...]
    col = lax.broadcasted_iota(jnp.int32, z.shape, 1)
    valid = col < 40
    zm = jnp.where(valid, z, NEG_INF)
    m = jnp.max(zm, axis=1, keepdims=True)
    ex = jnp.where(valid, jnp.exp(z - m), 0.0)
    s = jnp.sum(ex, axis=1, keepdims=True)
    res = z - m - jnp.log(s)
    out_ref[...] = res[:, :40]


def _log_softmax(z):
    n = z.shape[0]
    R = 1000
    return pl.pallas_call(
        _lsm_body,
        grid=(n // R,),
        in_specs=[pl.BlockSpec((R, 48), lambda i: (i, 0))],
        out_specs=pl.BlockSpec((R, 40), lambda i: (i, 0)),
        out_shape=jax.ShapeDtypeStruct((n, 40), jnp.float32),
    )(z)


def _make_edge(H, F, activate):
    NCH = F // 16
    hmap = [min((16 * i) // 64, H - 1) for i in range(NCH)]
    mesh = plsc.VectorSubcoreMesh(core_axis_name="c", subcore_axis_name="s",
                                  num_cores=2, num_subcores=16)

    @functools.partial(
        pl.kernel,
        out_type=jax.ShapeDtypeStruct((NP, F), jnp.float32),
        mesh=mesh,
        compiler_params=pltpu.CompilerParams(use_tc_tiling_on_sc=False),
        scratch_types=[
            pltpu.VMEM((NPT, F), jnp.float32),
            pltpu.VMEM((NPT, 16), jnp.float32),
            pltpu.VMEM((B,), jnp.int32),
            pltpu.VMEM((B,), jnp.int32),
            pltpu.VMEM((B, 16), jnp.float32),
            pltpu.VMEM((B, 16), jnp.float32),
            pltpu.VMEM((B, F), jnp.float32),
            pltpu.VMEM((NT, 16), jnp.int32),
            pltpu.SemaphoreType.DMA,
        ],
    )
    def edge_kernel(src_hbm, dst_hbm, el_hbm, er_hbm, feat_hbm, bnd_hbm,
                    out_hbm, acc, md, srcb, dstb, elb, erb, featb,
                    bvm, sem):
        lane = lax.broadcasted_iota(jnp.int32, (16,), 0)
        wid = lax.axis_index("c") * 16 + lax.axis_index("s")
        base = wid * NPT
        pltpu.sync_copy(bnd_hbm, bvm)
        brow = bvm[wid, :]
        start = brow[0]
        end = brow[1]
        start8 = (start // 8) * 8
        nblk = (end - start8 + B - 1) // B
        zero16 = lane.astype(jnp.float32) * 0.0
        mdinit = jnp.where(lane < 8, NEG_INF, 0.0)
        emask = lane < 8
        dmaskf = (jnp.where(lane >= 8, 1.0, 0.0)
                  * jnp.where(lane < 8 + H, 1.0, 0.0))
        ci = jnp.clip(lane - 8, 0, 3)

        def init_node(n0, _):
            for i in range(NCH):
                acc[n0, pl.ds(16 * i, 16)] = zero16
            md[n0, :] = mdinit
            return 0
        lax.fori_loop(0, NPT, init_node, 0)

        def load_block(e0, with_feat):
            pltpu.sync_copy(src_hbm.at[pl.ds(e0, B)], srcb)
            pltpu.sync_copy(dst_hbm.at[pl.ds(e0, B)], dstb)
            pltpu.async_copy(el_hbm.at[srcb], elb, sem).wait()
            pltpu.async_copy(er_hbm.at[dstb], erb, sem).wait()
            if with_feat:
                pltpu.async_copy(feat_hbm.at[srcb], featb, sem).wait()

        def b1_blk(k, _):
            e0 = start8 + k * B

            def b1_grp(g, _):
                o = g * 16
                dg = dstb[pl.ds(o, 16)]
                for jj in range(16):
                    eid = e0 + o + jj
                    valid = jnp.logical_and(eid >= start, eid < end)
                    dcl = jnp.clip(dg[jj] - base, 0, NPT - 1)
                    jr = o + jj
                    s16 = elb[jr, :] + erb[jr, :]
                    e16 = jnp.where(s16 >= 0, s16, NEG_SLOPE * s16)
                    e16v = e16 + jnp.where(valid, 0.0, NEG_INF)
                    row = md[dcl, :]
                    md[dcl, :] = jnp.where(emask, jnp.maximum(row, e16v),
                                           row)
                return 0

            load_block(e0, False)
            lax.fori_loop(0, B // 16, b1_grp, 0)
            return 0
        lax.fori_loop(0, nblk, b1_blk, 0)

        def c_blk(k, carry):
            e0 = start8 + k * B

            def c_grp(g, carry):
                d_prev, regs = carry
                o = g * 16
                dg = dstb[pl.ds(o, 16)]
                for jj in range(16):
                    eid = e0 + o + jj
                    valid = jnp.logical_and(eid >= start, eid < end)
                    validf = jnp.where(valid, 1.0, 0.0)
                    dcl = jnp.clip(dg[jj] - base, 0, NPT - 1)
                    d = jnp.where(valid, dcl, d_prev)
                    flush = d != d_prev

                    @pl.when(flush)
                    def _(dp=d_prev, rg=regs):
                        for i in range(NCH):
                            sl = pl.ds(16 * i, 16)
                            acc[dp, sl] = acc[dp, sl] + rg[i]

                    keepf = jnp.where(flush, 0.0, 1.0)
                    regs = tuple(r * keepf for r in regs)
                    jr = o + jj
                    s16 = elb[jr, :] + erb[jr, :]
                    e16 = jnp.where(s16 >= 0, s16, NEG_SLOPE * s16)
                    row = md[d, :]
                    ex16 = jnp.exp(jnp.minimum(e16 - row, 50.0)) * validf
                    md[d, :] = row + _gat16(ex16, ci) * dmaskf
                    exsp = [_splat16(ex16, h) for h in range(H)]
                    regs = tuple(
                        regs[i] + exsp[hmap[i]] * featb[jr, pl.ds(16 * i, 16)]
                        for i in range(NCH))
                    d_prev = d
                return d_prev, regs

            load_block(e0, True)
            return lax.fori_loop(0, B // 16, c_grp, carry)

        carry0 = (jnp.int32(0) * wid,
                  tuple(zero16 for _ in range(NCH)))
        d_prev, regs = lax.fori_loop(0, nblk, c_blk, carry0)
        for i in range(NCH):
            sl = pl.ds(16 * i, 16)
            acc[d_prev, sl] = acc[d_prev, sl] + regs[i]

        def d_node(n0, _):
            row = md[n0, :]
            rs = [1.0 / jnp.maximum(_splat16(row, 8 + h), 1e-9)
                  for h in range(H)]
            for i in range(NCH):
                sl = pl.ds(16 * i, 16)
                v = acc[n0, sl] * rs[hmap[i]]
                if activate:
                    v = jnp.where(v > 0, v, jnp.exp(v) - 1.0)
                acc[n0, sl] = v
            return 0
        lax.fori_loop(0, NPT, d_node, 0)
        pltpu.sync_copy(acc, out_hbm.at[pl.ds(base, NPT), :])

    return edge_kernel


_edge256 = _make_edge(4, 256, True)
_edge48 = _make_edge(1, 48, False)


def _amat(a, fo):
    ab = block_diag(*[a[h][:, None] for h in range(a.shape[0])])
    return jnp.pad(ab, ((0, fo - ab.shape[0]), (0, 16 - ab.shape[1])))


def kernel(x, edge_index, pred, conf, lg_s, node_s, epoch,
           W0, a0_src, a0_dst, W1, a1_src, a1_dst, W2, a2_src, a2_dst):
    src0 = edge_index[0].astype(jnp.int32)
    dst0 = edge_index[1].astype(jnp.int32)
    loops = jnp.arange(N, dtype=jnp.int32)
    src = jnp.concatenate([src0, dst0, loops])
    dst = jnp.concatenate([dst0, src0, loops])
    dst_s, src_s = lax.sort((dst, src), num_keys=1)
    bounds = jnp.searchsorted(
        dst_s, jnp.arange(33, dtype=jnp.int32) * NPT).astype(jnp.int32)
    b2 = jnp.stack([bounds[:32], bounds[1:33]], axis=1)
    b2 = jnp.pad(b2, ((0, 0), (0, 14)))
    pad = jnp.zeros((EPAD - ET,), jnp.int32)
    srcp = jnp.concatenate([src_s, pad])
    dstp = jnp.concatenate([dst_s, pad])

    A0s, A0d = _amat(a0_src, 256), _amat(a0_dst, 256)
    A1s, A1d = _amat(a1_src, 256), _amat(a1_dst, 256)
    A2s, A2d = _amat(a2_src, 48), _amat(a2_dst, 48)
    W2p = jnp.pad(W2, ((0, 0), (0, 8)))

    feat0, el0, er0 = _dense(x, W0, A0s, A0d, True)
    o0 = _edge256(srcp, dstp, el0, er0, feat0, b2)
    feat1, el1, er1 = _dense(o0[:N], W1, A1s, A1d, False)
    o1 = _edge256(srcp, dstp, el1, er1, feat1, b2)
    feat2, el2, er2 = _dense(o1[:N], W2p, A2s, A2d, False)
    o2 = _edge48(srcp, dstp, el2, er2, feat2, b2)
    return _log_softmax(o2[:N])


# speedup vs baseline: 36.0999x; 1.5356x over previous
import functools

import jax
import jax.numpy as jnp
from jax import lax
from jax.experimental import pallas as pl
from jax.experimental.pallas import tpu as pltpu
from jax.experimental.pallas import tpu_sc as plsc
from jax.scipy.linalg import block_diag

N = 10000
E = 320000
ET = 2 * E + N
NT = 32
NPT = 160
NR = 64
NP = NR * NPT
B = 128
EPAD = ET + 3 * B
NEG_SLOPE = 0.2
NEG_INF = float("-inf")


def _gat16(x, idx):
    dn = lax.GatherDimensionNumbers(
        offset_dims=(), collapsed_slice_dims=(0,), start_index_map=(0,))
    return lax.gather(x, idx[:, None], dn, slice_sizes=(1,),
                      mode=lax.GatherScatterMode.PROMISE_IN_BOUNDS)


def _splat16(x, h):
    idx = lax.broadcasted_iota(jnp.int32, (16,), 0) * 0 + h
    return _gat16(x, idx)


def _dense_body(h_ref, w_ref, asrc_ref, adst_ref, feat_ref, el_ref, er_ref,
                *, normalize):
    hb = h_ref[...]
    if normalize:
        s = jnp.sum(hb, axis=1, keepdims=True)
        hb = hb / jnp.maximum(s, 1.0)
    feat = jnp.dot(hb, w_ref[...], preferred_element_type=jnp.float32)
    feat_ref[...] = feat
    el_ref[...] = jnp.dot(feat, asrc_ref[...], preferred_element_type=jnp.float32)
    er_ref[...] = jnp.dot(feat, adst_ref[...], preferred_element_type=jnp.float32)


def _dense(h, W, Asrc, Adst, normalize):
    n, fi = h.shape
    fo = W.shape[1]
    R = 1000
    return pl.pallas_call(
        functools.partial(_dense_body, normalize=normalize),
        grid=(n // R,),
        in_specs=[
            pl.BlockSpec((R, fi), lambda i: (i, 0)),
            pl.BlockSpec((fi, fo), lambda i: (0, 0)),
            pl.BlockSpec((fo, 16), lambda i: (0, 0)),
            pl.BlockSpec((fo, 16), lambda i: (0, 0)),
        ],
        out_specs=[
            pl.BlockSpec((R, fo), lambda i: (i, 0)),
            pl.BlockSpec((R, 16), lambda i: (i, 0)),
            pl.BlockSpec((R, 16), lambda i: (i, 0)),
        ],
        out_shape=[
            jax.ShapeDtypeStruct((n, fo), jnp.float32),
            jax.ShapeDtypeStruct((n, 16), jnp.float32),
            jax.ShapeDtypeStruct((n, 16), jnp.float32),
        ],
    )(h, W, Asrc, Adst)


def _lsm_body(z_ref, out_ref):
    z = z_ref[...]
    col = lax.broadcasted_iota(jnp.int32, z.shape, 1)
    valid = col < 40
    zm = jnp.where(valid, z, NEG_INF)
    m = jnp.max(zm, axis=1, keepdims=True)
    ex = jnp.where(valid, jnp.exp(z - m), 0.0)
    s = jnp.sum(ex, axis=1, keepdims=True)
    res = z - m - jnp.log(s)
    out_ref[...] = res[:, :40]


def _log_softmax(z):
    n = z.shape[0]
    R = 1000
    return pl.pallas_call(
        _lsm_body,
        grid=(n // R,),
        in_specs=[pl.BlockSpec((R, 48), lambda i: (i, 0))],
        out_specs=pl.BlockSpec((R, 40), lambda i: (i, 0)),
        out_shape=jax.ShapeDtypeStruct((n, 40), jnp.float32),
    )(z)


def _make_edge(H, F, activate):
    NCH = F // 16
    hmap = [min((16 * i) // 64, H - 1) for i in range(NCH)]
    mesh = plsc.VectorSubcoreMesh(core_axis_name="c", subcore_axis_name="s",
                                  num_cores=2, num_subcores=16)

    @functools.partial(
        pl.kernel,
        out_type=jax.ShapeDtypeStruct((NP, F), jnp.float32),
        mesh=mesh,
        compiler_params=pltpu.CompilerParams(use_tc_tiling_on_sc=False),
        scratch_types=[
            pltpu.VMEM((NPT, F), jnp.float32),
            pltpu.VMEM((NPT, 16), jnp.float32),
            pltpu.VMEM((B,), jnp.int32),
            pltpu.VMEM((B,), jnp.int32),
            pltpu.VMEM((B,), jnp.int32),
            pltpu.VMEM((B,), jnp.int32),
            pltpu.VMEM((B, 16), jnp.float32),
            pltpu.VMEM((B, 16), jnp.float32),
            pltpu.VMEM((B, 16), jnp.float32),
            pltpu.VMEM((B, 16), jnp.float32),
            pltpu.VMEM((B, F), jnp.float32),
            pltpu.VMEM((B, F), jnp.float32),
            pltpu.VMEM((NR, 16), jnp.int32),
            pltpu.SemaphoreType.DMA,
            pltpu.SemaphoreType.DMA,
        ],
    )
    def edge_kernel(src_hbm, dst_hbm, el_hbm, er_hbm, feat_hbm, bnd_hbm,
                    out_hbm, acc, md, srcb0, srcb1, dstb0, dstb1,
                    elb0, elb1, erb0, erb1, featb0, featb1,
                    bvm, sem0, sem1):
        lane = lax.broadcasted_iota(jnp.int32, (16,), 0)
        wid = lax.axis_index("c") * 16 + lax.axis_index("s")
        pltpu.sync_copy(bnd_hbm, bvm)
        zero16 = lane.astype(jnp.float32) * 0.0
        mdinit = jnp.where(lane < 8, NEG_INF, 0.0)
        emask = lane < 8
        dmaskf = (jnp.where(lane >= 8, 1.0, 0.0)
                  * jnp.where(lane < 8 + H, 1.0, 0.0))
        ci = jnp.clip(lane - 8, 0, 3)
        srcbs = (srcb0, srcb1)
        dstbs = (dstb0, dstb1)
        elbs = (elb0, elb1)
        erbs = (erb0, erb1)
        featbs = (featb0, featb1)
        sems = (sem0, sem1)

        for half in range(2):
            rid = wid * 2 + half
            base = rid * NPT
            brow = bvm[rid, :]
            start = brow[0]
            end = brow[1]
            start8 = (start // 8) * 8
            nblk = (end - start8 + B - 1) // B
            nblkp = jnp.maximum((nblk + 1) // 2, 1)

            def init_node(n0, _):
                for i in range(NCH):
                    acc[n0, pl.ds(16 * i, 16)] = zero16
                md[n0, :] = mdinit
                return 0
            lax.fori_loop(0, NPT, init_node, 0)

            def issue(k, p, with_feat):
                e0 = start8 + k * B
                pltpu.sync_copy(src_hbm.at[pl.ds(e0, B)], srcbs[p])
                pltpu.sync_copy(dst_hbm.at[pl.ds(e0, B)], dstbs[p])
                pltpu.async_copy(el_hbm.at[srcbs[p]], elbs[p], sems[p])
                pltpu.async_copy(er_hbm.at[dstbs[p]], erbs[p], sems[p])
                if with_feat:
                    pltpu.async_copy(feat_hbm.at[srcbs[p]], featbs[p],
                                     sems[p])

            def drain(p, with_feat):
                pltpu.make_async_copy(el_hbm.at[pl.ds(0, B), :], elbs[p],
                                      sems[p]).wait()
                pltpu.make_async_copy(er_hbm.at[pl.ds(0, B), :], erbs[p],
                                      sems[p]).wait()
                if with_feat:
                    pltpu.make_async_copy(feat_hbm.at[pl.ds(0, B), :],
                                          featbs[p], sems[p]).wait()

            def b1_process(k, p):
                e0 = start8 + k * B

                def b1_grp(g, _):
                    o = g * 16
                    dg = dstbs[p][pl.ds(o, 16)]
                    for jj in range(16):
                        eid = e0 + o + jj
                        valid = jnp.logical_and(eid >= start, eid < end)
                        dcl = jnp.clip(dg[jj] - base, 0, NPT - 1)
                        jr = o + jj
                        s16 = elbs[p][jr, :] + erbs[p][jr, :]
                        e16 = jnp.where(s16 >= 0, s16, NEG_SLOPE * s16)
                        e16v = e16 + jnp.where(valid, 0.0, NEG_INF)
                        row = md[dcl, :]
                        md[dcl, :] = jnp.where(
                            emask, jnp.maximum(row, e16v), row)
                    return 0
                lax.fori_loop(0, B // 16, b1_grp, 0)

            issue(0, 0, False)

            def b1_pair(kk, _):
                k0 = 2 * kk
                issue(k0 + 1, 1, False)
                drain(0, False)
                b1_process(k0, 0)

                @pl.when(kk + 1 < nblkp)
                def _():
                    issue(k0 + 2, 0, False)

                drain(1, False)
                b1_process(k0 + 1, 1)
                return 0
            lax.fori_loop(0, nblkp, b1_pair, 0)

            def c_process(k, p, carry):
                e0 = start8 + k * B

                def c_grp(g, carry):
                    d_prev, regs = carry
                    o = g * 16
                    dg = dstbs[p][pl.ds(o, 16)]
                    for jj in range(16):
                        eid = e0 + o + jj
                        valid = jnp.logical_and(eid >= start, eid < end)
                        validf = jnp.where(valid, 1.0, 0.0)
                        dcl = jnp.clip(dg[jj] - base, 0, NPT - 1)
                        d = jnp.where(valid, dcl, d_prev)
                        flush = d != d_prev

                        @pl.when(flush)
                        def _(dp=d_prev, rg=regs):
                            for i in range(NCH):
                                sl = pl.ds(16 * i, 16)
                                acc[dp, sl] = acc[dp, sl] + rg[i]

                        keepf = jnp.where(flush, 0.0, 1.0)
                        regs = tuple(r * keepf for r in regs)
                        jr = o + jj
                        s16 = elbs[p][jr, :] + erbs[p][jr, :]
                        e16 = jnp.where(s16 >= 0, s16, NEG_SLOPE * s16)
                        row = md[d, :]
                        ex16 = jnp.exp(jnp.minimum(e16 - row, 50.0)) * validf
                        md[d, :] = row + _gat16(ex16, ci) * dmaskf
                        exsp = [_splat16(ex16, h) for h in range(H)]
                        regs = tuple(
                            regs[i]
                            + exsp[hmap[i]] * featbs[p][jr, pl.ds(16 * i, 16)]
                            for i in range(NCH))
                        d_prev = d
                    return d_prev, regs
                return lax.fori_loop(0, B // 16, c_grp, carry)

            issue(0, 0, True)

            def c_pair(kk, carry):
                k0 = 2 * kk
                issue(k0 + 1, 1, True)
                drain(0, True)
                carry = c_process(k0, 0, carry)

                @pl.when(kk + 1 < nblkp)
                def _():
                    issue(k0 + 2, 0, True)

                drain(1, True)
                carry = c_process(k0 + 1, 1, carry)
                return carry

            carry0 = (jnp.int32(0) * wid,
                      tuple(zero16 for _ in range(NCH)))
            d_prev, regs = lax.fori_loop(0, nblkp, c_pair, carry0)
            for i in range(NCH):
                sl = pl.ds(16 * i, 16)
                acc[d_prev, sl] = acc[d_prev, sl] + regs[i]

            def d_node(n0, _):
                row = md[n0, :]
                rs = [1.0 / jnp.maximum(_splat16(row, 8 + h), 1e-9)
                      for h in range(H)]
                for i in range(NCH):
                    sl = pl.ds(16 * i, 16)
                    v = acc[n0, sl] * rs[hmap[i]]
                    if activate:
                        v = jnp.where(v > 0, v, jnp.exp(v) - 1.0)
                    acc[n0, sl] = v
                return 0
            lax.fori_loop(0, NPT, d_node, 0)
            pltpu.sync_copy(acc, out_hbm.at[pl.ds(base, NPT), :])

    return edge_kernel


_edge256 = _make_edge(4, 256, True)
_edge48 = _make_edge(1, 48, False)


def _amat(a, fo):
    ab = block_diag(*[a[h][:, None] for h in range(a.shape[0])])
    return jnp.pad(ab, ((0, fo - ab.shape[0]), (0, 16 - ab.shape[1])))


def kernel(x, edge_index, pred, conf, lg_s, node_s, epoch,
           W0, a0_src, a0_dst, W1, a1_src, a1_dst, W2, a2_src, a2_dst):
    src0 = edge_index[0].astype(jnp.int32)
    dst0 = edge_index[1].astype(jnp.int32)
    loops = jnp.arange(N, dtype=jnp.int32)
    src = jnp.concatenate([src0, dst0, loops])
    dst = jnp.concatenate([dst0, src0, loops])
    dst_s, src_s = lax.sort((dst, src), num_keys=1)
    bounds = jnp.searchsorted(
        dst_s, jnp.arange(NR + 1, dtype=jnp.int32) * NPT).astype(jnp.int32)
    b2 = jnp.stack([bounds[:NR], bounds[1:NR + 1]], axis=1)
    b2 = jnp.pad(b2, ((0, 0), (0, 14)))
    pad = jnp.zeros((EPAD - ET,), jnp.int32)
    srcp = jnp.concatenate([src_s, pad])
    dstp = jnp.concatenate([dst_s, pad])

    A0s, A0d = _amat(a0_src, 256), _amat(a0_dst, 256)
    A1s, A1d = _amat(a1_src, 256), _amat(a1_dst, 256)
    A2s, A2d = _amat(a2_src, 48), _amat(a2_dst, 48)
    W2p = jnp.pad(W2, ((0, 0), (0, 8)))

    feat0, el0, er0 = _dense(x, W0, A0s, A0d, True)
    o0 = _edge256(srcp, dstp, el0, er0, feat0, b2)
    feat1, el1, er1 = _dense(o0[:N], W1, A1s, A1d, False)
    o1 = _edge256(srcp, dstp, el1, er1, feat1, b2)
    feat2, el2, er2 = _dense(o1[:N], W2p, A2s, A2d, False)
    o2 = _edge48(srcp, dstp, el2, er2, feat2, b2)
    return _log_softmax(o2[:N])


# speedup vs baseline: 39.1278x; 1.0839x over previous
import functools

import jax
import jax.numpy as jnp
from jax import lax
from jax.experimental import pallas as pl
from jax.experimental.pallas import tpu as pltpu
from jax.experimental.pallas import tpu_sc as plsc
from jax.scipy.linalg import block_diag

N = 10000
E = 320000
ET = 2 * E + N
NT = 32
NPT = 160
NR = 64
NP = NR * NPT
B = 128
EPAD = ET + 3 * B
NEG_SLOPE = 0.2
NEG_INF = float("-inf")


def _gat16(x, idx):
    dn = lax.GatherDimensionNumbers(
        offset_dims=(), collapsed_slice_dims=(0,), start_index_map=(0,))
    return lax.gather(x, idx[:, None], dn, slice_sizes=(1,),
                      mode=lax.GatherScatterMode.PROMISE_IN_BOUNDS)


def _splat16(x, h):
    idx = lax.broadcasted_iota(jnp.int32, (16,), 0) * 0 + h
    return _gat16(x, idx)


def _dense_body(h_ref, w_ref, asrc_ref, adst_ref, feat_ref, el_ref, er_ref,
                *, normalize):
    hb = h_ref[...]
    if normalize:
        s = jnp.sum(hb, axis=1, keepdims=True)
        hb = hb / jnp.maximum(s, 1.0)
    feat = jnp.dot(hb, w_ref[...], preferred_element_type=jnp.float32)
    feat_ref[...] = feat
    el_ref[...] = jnp.dot(feat, asrc_ref[...], preferred_element_type=jnp.float32)
    er_ref[...] = jnp.dot(feat, adst_ref[...], preferred_element_type=jnp.float32)


def _dense(h, W, Asrc, Adst, normalize):
    n, fi = h.shape
    fo = W.shape[1]
    R = 1000
    return pl.pallas_call(
        functools.partial(_dense_body, normalize=normalize),
        grid=(n // R,),
        in_specs=[
            pl.BlockSpec((R, fi), lambda i: (i, 0)),
            pl.BlockSpec((fi, fo), lambda i: (0, 0)),
            pl.BlockSpec((fo, 16), lambda i: (0, 0)),
            pl.BlockSpec((fo, 16), lambda i: (0, 0)),
        ],
        out_specs=[
            pl.BlockSpec((R, fo), lambda i: (i, 0)),
            pl.BlockSpec((R, 16), lambda i: (i, 0)),
            pl.BlockSpec((R, 16), lambda i: (i, 0)),
        ],
        out_shape=[
            jax.ShapeDtypeStruct((n, fo), jnp.float32),
            jax.ShapeDtypeStruct((n, 16), jnp.float32),
            jax.ShapeDtypeStruct((n, 16), jnp.float32),
        ],
    )(h, W, Asrc, Adst)


def _lsm_body(z_ref, out_ref):
    z = z_ref[...]
    col = lax.broadcasted_iota(jnp.int32, z.shape, 1)
    valid = col < 40
    zm = jnp.where(valid, z, NEG_INF)
    m = jnp.max(zm, axis=1, keepdims=True)
    ex = jnp.where(valid, jnp.exp(z - m), 0.0)
    s = jnp.sum(ex, axis=1, keepdims=True)
    res = z - m - jnp.log(s)
    out_ref[...] = res[:, :40]


def _log_softmax(z):
    n = z.shape[0]
    R = 1000
    return pl.pallas_call(
        _lsm_body,
        grid=(n // R,),
        in_specs=[pl.BlockSpec((R, 48), lambda i: (i, 0))],
        out_specs=pl.BlockSpec((R, 40), lambda i: (i, 0)),
        out_shape=jax.ShapeDtypeStruct((n, 40), jnp.float32),
    )(z)


def _make_edge(H, F, activate):
    NCH = F // 16
    hmap = [min((16 * i) // 64, H - 1) for i in range(NCH)]
    mesh = plsc.VectorSubcoreMesh(core_axis_name="c", subcore_axis_name="s",
                                  num_cores=2, num_subcores=16)

    @functools.partial(
        pl.kernel,
        out_type=jax.ShapeDtypeStruct((NP, F), jnp.float32),
        mesh=mesh,
        compiler_params=pltpu.CompilerParams(use_tc_tiling_on_sc=False),
        scratch_types=[
            pltpu.VMEM((NPT, F), jnp.float32),
            pltpu.VMEM((NPT, 16), jnp.float32),
            pltpu.VMEM((B,), jnp.int32),
            pltpu.VMEM((B,), jnp.int32),
            pltpu.VMEM((B,), jnp.int32),
            pltpu.VMEM((B,), jnp.int32),
            pltpu.VMEM((B, 16), jnp.float32),
            pltpu.VMEM((B, 16), jnp.float32),
            pltpu.VMEM((B, 16), jnp.float32),
            pltpu.VMEM((B, 16), jnp.float32),
            pltpu.VMEM((B, F), jnp.float32),
            pltpu.VMEM((B, F), jnp.float32),
            pltpu.VMEM((NR, 16), jnp.int32),
            pltpu.SemaphoreType.DMA,
            pltpu.SemaphoreType.DMA,
        ],
    )
    def edge_kernel(src_hbm, dst_hbm, el_hbm, er_hbm, feat_hbm, bnd_hbm,
                    out_hbm, acc, md, srcb0, srcb1, dstb0, dstb1,
                    elb0, elb1, erb0, erb1, featb0, featb1,
                    bvm, sem0, sem1):
        lane = lax.broadcasted_iota(jnp.int32, (16,), 0)
        wid = lax.axis_index("c") * 16 + lax.axis_index("s")
        pltpu.sync_copy(bnd_hbm, bvm)
        zero16 = lane.astype(jnp.float32) * 0.0
        mdinit = jnp.where(lane < 8, NEG_INF, 0.0)
        emask = lane < 8
        dmaskf = (jnp.where(lane >= 8, 1.0, 0.0)
                  * jnp.where(lane < 8 + H, 1.0, 0.0))
        ci = jnp.clip(lane - 8, 0, 3)
        srcbs = (srcb0, srcb1)
        dstbs = (dstb0, dstb1)
        elbs = (elb0, elb1)
        erbs = (erb0, erb1)
        featbs = (featb0, featb1)
        sems = (sem0, sem1)

        for half in range(2):
            rid = wid * 2 + half
            base = rid * NPT
            brow = bvm[rid, :]
            start = brow[0]
            end = brow[1]
            start8 = (start // 8) * 8
            nblk = (end - start8 + B - 1) // B
            nblkp = jnp.maximum((nblk + 1) // 2, 1)

            def init_node(n0, _):
                for i in range(NCH):
                    acc[n0, pl.ds(16 * i, 16)] = zero16
                md[n0, :] = mdinit
                return 0
            lax.fori_loop(0, NPT, init_node, 0)

            def issue(k, p, with_feat):
                e0 = start8 + k * B
                pltpu.sync_copy(src_hbm.at[pl.ds(e0, B)], srcbs[p])
                pltpu.sync_copy(dst_hbm.at[pl.ds(e0, B)], dstbs[p])
                pltpu.async_copy(el_hbm.at[srcbs[p]], elbs[p], sems[p])
                pltpu.async_copy(er_hbm.at[dstbs[p]], erbs[p], sems[p])
                if with_feat:
                    pltpu.async_copy(feat_hbm.at[srcbs[p]], featbs[p],
                                     sems[p])

            def drain(p, with_feat):
                pltpu.make_async_copy(el_hbm.at[pl.ds(0, B), :], elbs[p],
                                      sems[p]).wait()
                pltpu.make_async_copy(er_hbm.at[pl.ds(0, B), :], erbs[p],
                                      sems[p]).wait()
                if with_feat:
                    pltpu.make_async_copy(feat_hbm.at[pl.ds(0, B), :],
                                          featbs[p], sems[p]).wait()

            def b1_process(k, p, carry):
                e0 = start8 + k * B

                def b1_grp(g, carry):
                    d_prev, mrow = carry
                    o = g * 16
                    dg = dstbs[p][pl.ds(o, 16)]
                    for jj in range(16):
                        eid = e0 + o + jj
                        valid = jnp.logical_and(eid >= start, eid < end)
                        dcl = jnp.clip(dg[jj] - base, 0, NPT - 1)
                        d = jnp.where(valid, dcl, d_prev)
                        flush = d != d_prev

                        @pl.when(flush)
                        def _(dp=d_prev, mr=mrow):
                            md[dp, :] = mr

                        mrow = jnp.where(flush, mdinit, mrow)
                        jr = o + jj
                        s16 = elbs[p][jr, :] + erbs[p][jr, :]
                        e16 = jnp.where(s16 >= 0, s16, NEG_SLOPE * s16)
                        e16v = e16 + jnp.where(valid, 0.0, NEG_INF)
                        mrow = jnp.where(emask, jnp.maximum(mrow, e16v),
                                         mrow)
                        d_prev = d
                    return d_prev, mrow
                return lax.fori_loop(0, B // 16, b1_grp, carry)

            issue(0, 0, False)

            def b1_pair(kk, carry):
                k0 = 2 * kk
                issue(k0 + 1, 1, False)
                drain(0, False)
                carry = b1_process(k0, 0, carry)

                @pl.when(kk + 1 < nblkp)
                def _():
                    issue(k0 + 2, 0, False)

                drain(1, False)
                carry = b1_process(k0 + 1, 1, carry)
                return carry
            bcarry0 = (jnp.int32(0) * wid, mdinit)
            bd_prev, bmrow = lax.fori_loop(0, nblkp, b1_pair, bcarry0)
            md[bd_prev, :] = bmrow

            def c_process(k, p, carry):
                e0 = start8 + k * B

                def c_grp(g, carry):
                    d_prev, mrow, regs = carry
                    o = g * 16
                    dg = dstbs[p][pl.ds(o, 16)]
                    for jj in range(16):
                        eid = e0 + o + jj
                        valid = jnp.logical_and(eid >= start, eid < end)
                        validf = jnp.where(valid, 1.0, 0.0)
                        dcl = jnp.clip(dg[jj] - base, 0, NPT - 1)
                        d = jnp.where(valid, dcl, d_prev)
                        flush = d != d_prev

                        @pl.when(flush)
                        def _(dp=d_prev, rg=regs, mr=mrow):
                            md[dp, :] = mr
                            for i in range(NCH):
                                sl = pl.ds(16 * i, 16)
                                acc[dp, sl] = acc[dp, sl] + rg[i]

                        keepf = jnp.where(flush, 0.0, 1.0)
                        regs = tuple(r * keepf for r in regs)
                        md_d = md[d, :]
                        mrow = jnp.where(flush, md_d, mrow)
                        jr = o + jj
                        s16 = elbs[p][jr, :] + erbs[p][jr, :]
                        e16 = jnp.where(s16 >= 0, s16, NEG_SLOPE * s16)
                        ex16 = (jnp.exp(jnp.minimum(e16 - mrow, 50.0))
                                * validf)
                        mrow = mrow + _gat16(ex16, ci) * dmaskf
                        exsp = [_splat16(ex16, h) for h in range(H)]
                        regs = tuple(
                            regs[i]
                            + exsp[hmap[i]] * featbs[p][jr, pl.ds(16 * i, 16)]
                            for i in range(NCH))
                        d_prev = d
                    return d_prev, mrow, regs
                return lax.fori_loop(0, B // 16, c_grp, carry)

            issue(0, 0, True)

            def c_pair(kk, carry):
                k0 = 2 * kk
                issue(k0 + 1, 1, True)
                drain(0, True)
                carry = c_process(k0, 0, carry)

                @pl.when(kk + 1 < nblkp)
                def _():
                    issue(k0 + 2, 0, True)

                drain(1, True)
                carry = c_process(k0 + 1, 1, carry)
                return carry

            carry0 = (jnp.int32(0) * wid, md[0, :],
                      tuple(zero16 for _ in range(NCH)))
            d_prev, mrowf, regs = lax.fori_loop(0, nblkp, c_pair, carry0)
            md[d_prev, :] = mrowf
            for i in range(NCH):
                sl = pl.ds(16 * i, 16)
                acc[d_prev, sl] = acc[d_prev, sl] + regs[i]

            def d_node(n0, _):
                row = md[n0, :]
                rs = [1.0 / jnp.maximum(_splat16(row, 8 + h), 1e-9)
                      for h in range(H)]
                for i in range(NCH):
                    sl = pl.ds(16 * i, 16)
                    v = acc[n0, sl] * rs[hmap[i]]
                    if activate:
                        v = jnp.where(v > 0, v, jnp.exp(v) - 1.0)
                    acc[n0, sl] = v
                return 0
            lax.fori_loop(0, NPT, d_node, 0)
            pltpu.sync_copy(acc, out_hbm.at[pl.ds(base, NPT), :])

    return edge_kernel


_edge256 = _make_edge(4, 256, True)
_edge48 = _make_edge(1, 48, False)


def _amat(a, fo):
    ab = block_diag(*[a[h][:, None] for h in range(a.shape[0])])
    return jnp.pad(ab, ((0, fo - ab.shape[0]), (0, 16 - ab.shape[1])))


def kernel(x, edge_index, pred, conf, lg_s, node_s, epoch,
           W0, a0_src, a0_dst, W1, a1_src, a1_dst, W2, a2_src, a2_dst):
    src0 = edge_index[0].astype(jnp.int32)
    dst0 = edge_index[1].astype(jnp.int32)
    loops = jnp.arange(N, dtype=jnp.int32)
    src = jnp.concatenate([src0, dst0, loops])
    dst = jnp.concatenate([dst0, src0, loops])
    dst_s, src_s = lax.sort((dst, src), num_keys=1)
    bounds = jnp.searchsorted(
        dst_s, jnp.arange(NR + 1, dtype=jnp.int32) * NPT).astype(jnp.int32)
    b2 = jnp.stack([bounds[:NR], bounds[1:NR + 1]], axis=1)
    b2 = jnp.pad(b2, ((0, 0), (0, 14)))
    pad = jnp.zeros((EPAD - ET,), jnp.int32)
    srcp = jnp.concatenate([src_s, pad])
    dstp = jnp.concatenate([dst_s, pad])

    A0s, A0d = _amat(a0_src, 256), _amat(a0_dst, 256)
    A1s, A1d = _amat(a1_src, 256), _amat(a1_dst, 256)
    A2s, A2d = _amat(a2_src, 48), _amat(a2_dst, 48)
    W2p = jnp.pad(W2, ((0, 0), (0, 8)))

    feat0, el0, er0 = _dense(x, W0, A0s, A0d, True)
    o0 = _edge256(srcp, dstp, el0, er0, feat0, b2)
    feat1, el1, er1 = _dense(o0[:N], W1, A1s, A1d, False)
    o1 = _edge256(srcp, dstp, el1, er1, feat1, b2)
    feat2, el2, er2 = _dense(o1[:N], W2p, A2s, A2d, False)
    o2 = _edge48(srcp, dstp, el2, er2, feat2, b2)
    return _log_softmax(o2[:N])


# speedup vs baseline: 50.9997x; 1.3034x over previous
import functools

import jax
import jax.numpy as jnp
from jax import lax
from jax.experimental import pallas as pl
from jax.experimental.pallas import tpu as pltpu
from jax.experimental.pallas import tpu_sc as plsc
from jax.scipy.linalg import block_diag

N = 10000
E = 320000
ET = 2 * E + N
NT = 32
NPT = 160
NR = 64
NP = NR * NPT
B = 128
EPAD = ET + 3 * B
NEG_SLOPE = 0.2
NEG_INF = float("-inf")


def _gat16(x, idx):
    dn = lax.GatherDimensionNumbers(
        offset_dims=(), collapsed_slice_dims=(0,), start_index_map=(0,))
    return lax.gather(x, idx[:, None], dn, slice_sizes=(1,),
                      mode=lax.GatherScatterMode.PROMISE_IN_BOUNDS)


def _splat16(x, h):
    idx = lax.broadcasted_iota(jnp.int32, (16,), 0) * 0 + h
    return _gat16(x, idx)


def _dense_body(h_ref, w_ref, asrc_ref, adst_ref, feat_ref, el_ref, er_ref,
                elmax_ref, *, normalize):
    hb = h_ref[...]
    if normalize:
        s = jnp.sum(hb, axis=1, keepdims=True)
        hb = hb / jnp.maximum(s, 1.0)
    feat = jnp.dot(hb, w_ref[...], preferred_element_type=jnp.float32)
    feat_ref[...] = feat
    el = jnp.dot(feat, asrc_ref[...], preferred_element_type=jnp.float32)
    el_ref[...] = el
    er_ref[...] = jnp.dot(feat, adst_ref[...], preferred_element_type=jnp.float32)
    bm = jnp.max(el, axis=0, keepdims=True)
    i = pl.program_id(0)
    elmax_ref[...] = jnp.where(i == 0, bm,
                               jnp.maximum(elmax_ref[...], bm))


def _dense(h, W, Asrc, Adst, normalize):
    n, fi = h.shape
    fo = W.shape[1]
    R = 1000
    return pl.pallas_call(
        functools.partial(_dense_body, normalize=normalize),
        grid=(n // R,),
        in_specs=[
            pl.BlockSpec((R, fi), lambda i: (i, 0)),
            pl.BlockSpec((fi, fo), lambda i: (0, 0)),
            pl.BlockSpec((fo, 16), lambda i: (0, 0)),
            pl.BlockSpec((fo, 16), lambda i: (0, 0)),
        ],
        out_specs=[
            pl.BlockSpec((R, fo), lambda i: (i, 0)),
            pl.BlockSpec((R, 16), lambda i: (i, 0)),
            pl.BlockSpec((R, 16), lambda i: (i, 0)),
            pl.BlockSpec((1, 16), lambda i: (0, 0)),
        ],
        out_shape=[
            jax.ShapeDtypeStruct((n, fo), jnp.float32),
            jax.ShapeDtypeStruct((n, 16), jnp.float32),
            jax.ShapeDtypeStruct((n, 16), jnp.float32),
            jax.ShapeDtypeStruct((1, 16), jnp.float32),
        ],
    )(h, W, Asrc, Adst)


def _lsm_body(z_ref, out_ref):
    z = z_ref[...]
    col = lax.broadcasted_iota(jnp.int32, z.shape, 1)
    valid = col < 40
    zm = jnp.where(valid, z, NEG_INF)
    m = jnp.max(zm, axis=1, keepdims=True)
    ex = jnp.where(valid, jnp.exp(z - m), 0.0)
    s = jnp.sum(ex, axis=1, keepdims=True)
    res = z - m - jnp.log(s)
    out_ref[...] = res[:, :40]


def _log_softmax(z):
    n = z.shape[0]
    R = 1000
    return pl.pallas_call(
        _lsm_body,
        grid=(n // R,),
        in_specs=[pl.BlockSpec((R, 48), lambda i: (i, 0))],
        out_specs=pl.BlockSpec((R, 40), lambda i: (i, 0)),
        out_shape=jax.ShapeDtypeStruct((n, 40), jnp.float32),
    )(z)


def _make_edge(H, F, activate):
    NCH = F // 16
    hmap = [min((16 * i) // 64, H - 1) for i in range(NCH)]
    mesh = plsc.VectorSubcoreMesh(core_axis_name="c", subcore_axis_name="s",
                                  num_cores=2, num_subcores=16)

    @functools.partial(
        pl.kernel,
        out_type=jax.ShapeDtypeStruct((NP, F), jnp.float32),
        mesh=mesh,
        compiler_params=pltpu.CompilerParams(use_tc_tiling_on_sc=False),
        scratch_types=[
            pltpu.VMEM((NPT, F), jnp.float32),
            pltpu.VMEM((NPT, 16), jnp.float32),
            pltpu.VMEM((B,), jnp.int32),
            pltpu.VMEM((B,), jnp.int32),
            pltpu.VMEM((B,), jnp.int32),
            pltpu.VMEM((B,), jnp.int32),
            pltpu.VMEM((B, 16), jnp.float32),
            pltpu.VMEM((B, 16), jnp.float32),
            pltpu.VMEM((B, 16), jnp.float32),
            pltpu.VMEM((B, 16), jnp.float32),
            pltpu.VMEM((B, F), jnp.float32),
            pltpu.VMEM((B, F), jnp.float32),
            pltpu.VMEM((NR, 16), jnp.int32),
            pltpu.VMEM((NPT, 16), jnp.float32),
            pltpu.VMEM((16,), jnp.float32),
            pltpu.SemaphoreType.DMA,
            pltpu.SemaphoreType.DMA,
        ],
    )
    def edge_kernel(src_hbm, dst_hbm, el_hbm, er_hbm, feat_hbm, bnd_hbm,
                    elmax_hbm, out_hbm, acc, md, srcb0, srcb1, dstb0, dstb1,
                    elb0, elb1, erb0, erb1, featb0, featb1,
                    bvm, erloc, elmx, sem0, sem1):
        lane = lax.broadcasted_iota(jnp.int32, (16,), 0)
        wid = lax.axis_index("c") * 16 + lax.axis_index("s")
        pltpu.sync_copy(bnd_hbm, bvm)
        pltpu.sync_copy(elmax_hbm, elmx)
        zero16 = lane.astype(jnp.float32) * 0.0
        mdinit = jnp.where(lane < 8, NEG_INF, 0.0)
        emask = lane < 8
        dmaskf = (jnp.where(lane >= 8, 1.0, 0.0)
                  * jnp.where(lane < 8 + H, 1.0, 0.0))
        ci = jnp.clip(lane - 8, 0, 3)
        srcbs = (srcb0, srcb1)
        dstbs = (dstb0, dstb1)
        elbs = (elb0, elb1)
        erbs = (erb0, erb1)
        featbs = (featb0, featb1)
        sems = (sem0, sem1)

        for half in range(2):
            rid = wid * 2 + half
            base = rid * NPT
            brow = bvm[rid, :]
            start = brow[0]
            end = brow[1]
            start8 = (start // 8) * 8
            nblk = (end - start8 + B - 1) // B
            nblkp = jnp.maximum((nblk + 1) // 2, 1)

            pltpu.sync_copy(er_hbm.at[pl.ds(base, NPT), :], erloc)
            emx16 = elmx[pl.ds(0, 16)]

            def init_node(n0, _):
                for i in range(NCH):
                    acc[n0, pl.ds(16 * i, 16)] = zero16
                s16 = emx16 + erloc[n0, :]
                ub = jnp.where(s16 >= 0, s16, NEG_SLOPE * s16)
                md[n0, :] = jnp.where(emask, ub, 0.0)
                return 0
            lax.fori_loop(0, NPT, init_node, 0)

            def issue(k, p, with_feat):
                e0 = start8 + k * B
                pltpu.sync_copy(src_hbm.at[pl.ds(e0, B)], srcbs[p])
                pltpu.sync_copy(dst_hbm.at[pl.ds(e0, B)], dstbs[p])
                pltpu.async_copy(el_hbm.at[srcbs[p]], elbs[p], sems[p])
                pltpu.async_copy(er_hbm.at[dstbs[p]], erbs[p], sems[p])
                if with_feat:
                    pltpu.async_copy(feat_hbm.at[srcbs[p]], featbs[p],
                                     sems[p])

            def drain(p, with_feat):
                pltpu.make_async_copy(el_hbm.at[pl.ds(0, B), :], elbs[p],
                                      sems[p]).wait()
                pltpu.make_async_copy(er_hbm.at[pl.ds(0, B), :], erbs[p],
                                      sems[p]).wait()
                if with_feat:
                    pltpu.make_async_copy(feat_hbm.at[pl.ds(0, B), :],
                                          featbs[p], sems[p]).wait()

            def c_process(k, p, carry):
                e0 = start8 + k * B

                def c_grp(g, carry):
                    d_prev, mrow, regs = carry
                    o = g * 16
                    dg = dstbs[p][pl.ds(o, 16)]
                    for jj in range(16):
                        eid = e0 + o + jj
                        valid = jnp.logical_and(eid >= start, eid < end)
                        validf = jnp.where(valid, 1.0, 0.0)
                        dcl = jnp.clip(dg[jj] - base, 0, NPT - 1)
                        d = jnp.where(valid, dcl, d_prev)
                        flush = d != d_prev

                        @pl.when(flush)
                        def _(dp=d_prev, rg=regs, mr=mrow):
                            md[dp, :] = mr
                            for i in range(NCH):
                                sl = pl.ds(16 * i, 16)
                                acc[dp, sl] = acc[dp, sl] + rg[i]

                        keepf = jnp.where(flush, 0.0, 1.0)
                        regs = tuple(r * keepf for r in regs)
                        md_d = md[d, :]
                        mrow = jnp.where(flush, md_d, mrow)
                        jr = o + jj
                        s16 = elbs[p][jr, :] + erbs[p][jr, :]
                        e16 = jnp.where(s16 >= 0, s16, NEG_SLOPE * s16)
                        ex16 = (jnp.exp(jnp.minimum(e16 - mrow, 50.0))
                                * validf)
                        mrow = mrow + _gat16(ex16, ci) * dmaskf
                        exsp = [_splat16(ex16, h) for h in range(H)]
                        regs = tuple(
                            regs[i]
                            + exsp[hmap[i]] * featbs[p][jr, pl.ds(16 * i, 16)]
                            for i in range(NCH))
                        d_prev = d
                    return d_prev, mrow, regs
                return lax.fori_loop(0, B // 16, c_grp, carry)

            issue(0, 0, True)

            def c_pair(kk, carry):
                k0 = 2 * kk
                issue(k0 + 1, 1, True)
                drain(0, True)
                carry = c_process(k0, 0, carry)

                @pl.when(kk + 1 < nblkp)
                def _():
                    issue(k0 + 2, 0, True)

                drain(1, True)
                carry = c_process(k0 + 1, 1, carry)
                return carry

            carry0 = (jnp.int32(0) * wid, md[0, :],
                      tuple(zero16 for _ in range(NCH)))
            d_prev, mrowf, regs = lax.fori_loop(0, nblkp, c_pair, carry0)
            md[d_prev, :] = mrowf
            for i in range(NCH):
                sl = pl.ds(16 * i, 16)
                acc[d_prev, sl] = acc[d_prev, sl] + regs[i]

            def d_node(n0, _):
                row = md[n0, :]
                rs = [1.0 / jnp.maximum(_splat16(row, 8 + h), 1e-9)
                      for h in range(H)]
                for i in range(NCH):
                    sl = pl.ds(16 * i, 16)
                    v = acc[n0, sl] * rs[hmap[i]]
                    if activate:
                        v = jnp.where(v > 0, v, jnp.exp(v) - 1.0)
                    acc[n0, sl] = v
                return 0
            lax.fori_loop(0, NPT, d_node, 0)
            pltpu.sync_copy(acc, out_hbm.at[pl.ds(base, NPT), :])

    return edge_kernel


_edge256 = _make_edge(4, 256, True)
_edge48 = _make_edge(1, 48, False)


def _amat(a, fo):
    ab = block_diag(*[a[h][:, None] for h in range(a.shape[0])])
    return jnp.pad(ab, ((0, fo - ab.shape[0]), (0, 16 - ab.shape[1])))


def kernel(x, edge_index, pred, conf, lg_s, node_s, epoch,
           W0, a0_src, a0_dst, W1, a1_src, a1_dst, W2, a2_src, a2_dst):
    src0 = edge_index[0].astype(jnp.int32)
    dst0 = edge_index[1].astype(jnp.int32)
    loops = jnp.arange(N, dtype=jnp.int32)
    src = jnp.concatenate([src0, dst0, loops])
    dst = jnp.concatenate([dst0, src0, loops])
    dst_s, src_s = lax.sort((dst, src), num_keys=1)
    bounds = jnp.searchsorted(
        dst_s, jnp.arange(NR + 1, dtype=jnp.int32) * NPT).astype(jnp.int32)
    b2 = jnp.stack([bounds[:NR], bounds[1:NR + 1]], axis=1)
    b2 = jnp.pad(b2, ((0, 0), (0, 14)))
    pad = jnp.zeros((EPAD - ET,), jnp.int32)
    srcp = jnp.concatenate([src_s, pad])
    dstp = jnp.concatenate([dst_s, pad])

    A0s, A0d = _amat(a0_src, 256), _amat(a0_dst, 256)
    A1s, A1d = _amat(a1_src, 256), _amat(a1_dst, 256)
    A2s, A2d = _amat(a2_src, 48), _amat(a2_dst, 48)
    W2p = jnp.pad(W2, ((0, 0), (0, 8)))

    feat0, el0, er0, em0 = _dense(x, W0, A0s, A0d, True)
    o0 = _edge256(srcp, dstp, el0, er0, feat0, b2, em0[0])
    feat1, el1, er1, em1 = _dense(o0[:N], W1, A1s, A1d, False)
    o1 = _edge256(srcp, dstp, el1, er1, feat1, b2, em1[0])
    feat2, el2, er2, em2 = _dense(o1[:N], W2p, A2s, A2d, False)
    o2 = _edge48(srcp, dstp, el2, er2, feat2, b2, em2[0])
    return _log_softmax(o2[:N])
